# baseline (speedup 1.0000x reference)
import functools

import jax
import jax.numpy as jnp
from jax import lax
from jax.experimental import pallas as pl
from jax.experimental.pallas import tpu as pltpu
from jax.experimental.pallas import tpu_sc as plsc

_V, _D, _B = 1000000, 32, 16384


@functools.cache
def _build_lookup(V, D, B):
  info = plsc.get_sparse_core_info()
  NC, NS = info.num_cores, info.num_subcores
  NW = NC * NS
  assert B % (8 * NW) == 0
  b_per_w = B // NW
  mesh = plsc.VectorSubcoreMesh(core_axis_name="c", subcore_axis_name="s")

  @functools.partial(
      pl.kernel,
      out_type=[jax.ShapeDtypeStruct((B, D), jnp.float32)] * 4,
      mesh=mesh,
      compiler_params=pltpu.CompilerParams(use_tc_tiling_on_sc=False),
      scratch_types=[
          pltpu.VMEM((b_per_w,), jnp.int32),
          pltpu.VMEM((b_per_w,), jnp.int32),
          pltpu.VMEM((b_per_w, D), jnp.float32),
          pltpu.VMEM((b_per_w, D), jnp.float32),
          pltpu.VMEM((b_per_w, D), jnp.float32),
          pltpu.VMEM((b_per_w, D), jnp.float32),
          pltpu.SemaphoreType.DMA,
      ],
  )
  def lookup(emb_hbm, emb2_hbm, ctx_hbm, vi_hbm, vj_hbm,
             o1, o2, o3, o4, idx_i, idx_j, r1, r2, r3, r4, sem):
    wid = lax.axis_index("s") * NC + lax.axis_index("c")
    base = wid * b_per_w
    pltpu.sync_copy(vi_hbm.at[pl.ds(base, b_per_w)], idx_i)
    pltpu.sync_copy(vj_hbm.at[pl.ds(base, b_per_w)], idx_j)
    c1 = pltpu.async_copy(emb_hbm.at[idx_i], r1, sem)
    c2 = pltpu.async_copy(emb_hbm.at[idx_j], r2, sem)
    c3 = pltpu.async_copy(emb2_hbm.at[idx_i], r3, sem)
    c4 = pltpu.async_copy(ctx_hbm.at[idx_j], r4, sem)
    c1.wait()
    pltpu.sync_copy(r1, o1.at[pl.ds(base, b_per_w)])
    c2.wait()
    pltpu.sync_copy(r2, o2.at[pl.ds(base, b_per_w)])
    c3.wait()
    pltpu.sync_copy(r3, o3.at[pl.ds(base, b_per_w)])
    c4.wait()
    pltpu.sync_copy(r4, o4.at[pl.ds(base, b_per_w)])

  return lookup


def kernel(nodeindex, v_i, v_j, device, embeddings, second_embeddings,
           context_embeddings):
  lookup = _build_lookup(_V, _D, _B)
  u_i1, u_j1, u_i2, u_j2 = lookup(
      embeddings, second_embeddings, context_embeddings, v_i, v_j)
  return (u_i1, u_j1, u_i2, u_j2)


# speedup vs baseline: 2.9702x; 2.9702x over previous
import functools

import jax
import jax.numpy as jnp
from jax import lax
from jax.experimental import pallas as pl
from jax.experimental.pallas import tpu as pltpu
from jax.experimental.pallas import tpu_sc as plsc

_V, _D, _B = 1000000, 32, 16384
_NC, _NS = 2, 16
_NW = _NC * _NS
_C = 1024
_NSLAB_UNIFORM = 30
_N_EXTRA = 976 - 32 * _NSLAB_UNIFORM
_T1_OFF = 976 * _C
_T2_OFF = _T1_OFF + 512


def _worker_id():
  return lax.axis_index("s") * _NC + lax.axis_index("c")


@functools.cache
def _build_detile():
  mesh = plsc.VectorSubcoreMesh(core_axis_name="c", subcore_axis_name="s")
  bufw = _D * _C

  @functools.partial(
      pl.kernel,
      out_type=[jax.ShapeDtypeStruct((_D * _V,), jnp.float32)] * 3,
      mesh=mesh,
      compiler_params=pltpu.CompilerParams(use_tc_tiling_on_sc=True),
      scratch_types=[
          pltpu.VMEM((bufw,), jnp.float32),
          pltpu.VMEM((bufw,), jnp.float32),
          pltpu.SemaphoreType.DMA,
          pltpu.SemaphoreType.DMA,
          pltpu.SemaphoreType.DMA,
          pltpu.SemaphoreType.DMA,
      ],
  )
  def detile(t1, t2, t3, tail, s1, s2, s3,
             bufa, bufb, in_a, in_b, out_a, out_b):
    wid = _worker_id()
    bufs = (bufa, bufb)
    in_sems = (in_a, in_b)
    out_sems = (out_a, out_b)

    def fire_in(src, col, width, buf, sem):
      for c in range(_D):
        pltpu.async_copy(src.at[c, pl.ds(col, width)],
                         buf.at[pl.ds(c * _C, width)], sem)

    def fire_out(dst, col, width, buf, sem):
      for c in range(_D):
        pltpu.async_copy(buf.at[pl.ds(c * _C, width)],
                         dst.at[pl.ds(c * _V + col, width)], sem)

    def drain(src, words, sem):
      pltpu.make_async_copy(src.at[0, pl.ds(0, words)],
                            bufa.at[pl.ds(0, words)], sem).wait()

    for src, dst in ((t1, s1), (t2, s2), (t3, s3)):
      def slab(k, _, src=src, dst=dst):
        col = (wid * _NSLAB_UNIFORM + k) * _C
        for par in (0, 1):
          @pl.when(k % 2 == par)
          def _(par=par):
            buf = bufs[par]

            @pl.when(k >= 2)
            def _():
              drain(src, _D * _C, out_sems[par])
            fire_in(src, col, _C, buf, in_sems[par])
            drain(src, _D * _C, in_sems[par])
            fire_out(dst, col, _C, buf, out_sems[par])
        return 0

      lax.fori_loop(0, _NSLAB_UNIFORM, slab, 0)
      drain(src, _D * _C, out_sems[_NSLAB_UNIFORM % 2])
      drain(src, _D * _C, out_sems[(_NSLAB_UNIFORM + 1) % 2])

      @pl.when(wid < _N_EXTRA)
      def _(src=src, dst=dst):
        col = (960 + wid) * _C
        fire_in(src, col, _C, bufa, in_a)
        drain(src, _D * _C, in_a)
        fire_out(dst, col, _C, bufa, out_a)
        drain(src, _D * _C, out_a)

      @pl.when(wid == 30)
      def _(src=src, dst=dst):
        fire_in(src, _T1_OFF, 512, bufa, in_a)
        drain(src, _D * 512, in_a)
        fire_out(dst, _T1_OFF, 512, bufa, out_a)
        drain(src, _D * 512, out_a)

    @pl.when(wid == 31)
    def _():
      pltpu.sync_copy(tail, bufb.at[pl.ds(0, 3 * _D * 64)])
      for i, dst in enumerate((s1, s2, s3)):
        for c in range(_D):
          pltpu.async_copy(bufb.at[pl.ds((i * _D + c) * 64, 64)],
                           dst.at[pl.ds(c * _V + _T2_OFF, 64)], out_b)
      drain(t1, 3 * _D * 64, out_b)

  return detile


@functools.cache
def _build_gather():
  mesh = plsc.VectorSubcoreMesh(core_axis_name="c", subcore_axis_name="s")
  nw_words = (_B // _NW) * _D

  @functools.partial(
      pl.kernel,
      out_type=[jax.ShapeDtypeStruct((_B * _D,), jnp.float32)] * 4,
      mesh=mesh,
      compiler_params=pltpu.CompilerParams(use_tc_tiling_on_sc=False),
      scratch_types=[
          pltpu.VMEM((nw_words,), jnp.int32),
          pltpu.VMEM((nw_words,), jnp.int32),
          pltpu.VMEM((nw_words,), jnp.float32),
          pltpu.VMEM((nw_words,), jnp.float32),
          pltpu.VMEM((nw_words,), jnp.float32),
          pltpu.VMEM((nw_words,), jnp.float32),
          pltpu.SemaphoreType.DMA,
      ],
  )
  def gather(s1f, s2f, s3f, ei_hbm, ej_hbm,
             o1, o2, o3, o4, ei, ej, r1, r2, r3, r4, sem):
    wid = _worker_id()
    base = wid * nw_words
    pltpu.sync_copy(ei_hbm.at[pl.ds(base, nw_words)], ei)
    pltpu.sync_copy(ej_hbm.at[pl.ds(base, nw_words)], ej)
    c1 = pltpu.async_copy(s1f.at[ei], r1, sem)
    c2 = pltpu.async_copy(s1f.at[ej], r2, sem)
    c3 = pltpu.async_copy(s2f.at[ei], r3, sem)
    c4 = pltpu.async_copy(s3f.at[ej], r4, sem)
    c1.wait()
    pltpu.sync_copy(r1, o1.at[pl.ds(base, nw_words)])
    c2.wait()
    pltpu.sync_copy(r2, o2.at[pl.ds(base, nw_words)])
    c3.wait()
    pltpu.sync_copy(r3, o3.at[pl.ds(base, nw_words)])
    c4.wait()
    pltpu.sync_copy(r4, o4.at[pl.ds(base, nw_words)])

  return gather


def kernel(nodeindex, v_i, v_j, device, embeddings, second_embeddings,
           context_embeddings):
  detile = _build_detile()
  gather = _build_gather()

  tails = jnp.concatenate(
      [t[_T2_OFF:, :].T.reshape(-1)
       for t in (embeddings, second_embeddings, context_embeddings)])
  s1f, s2f, s3f = detile(embeddings.T, second_embeddings.T,
                         context_embeddings.T, tails)

  coff = (jnp.arange(_D, dtype=jnp.int32) * _V)[None, :]
  ei = jnp.reshape(v_i.astype(jnp.int32)[:, None] + coff, (_B * _D,))
  ej = jnp.reshape(v_j.astype(jnp.int32)[:, None] + coff, (_B * _D,))

  f1, f2, f3, f4 = gather(s1f, s2f, s3f, ei, ej)
  return (jnp.reshape(f1, (_B, _D)), jnp.reshape(f2, (_B, _D)),
          jnp.reshape(f3, (_B, _D)), jnp.reshape(f4, (_B, _D)))
